# Initial kernel scaffold; baseline (speedup 1.0000x reference)
import jax
import jax.numpy as jnp
from jax.experimental import pallas as pl


def kernel(target, context, noise, in_embed, out_embed):
    raise NotImplementedError("write your pallas kernel here")


# speedup vs baseline: 3.9114x; 3.9114x over previous
import functools

import jax
import jax.numpy as jnp
from jax import lax
from jax.experimental import pallas as pl
from jax.experimental.pallas import tpu as pltpu
from jax.experimental.pallas import tpu_sc as plsc

V = 1000000
D = 64
B = 16384
K = 20

NW = 32
BPW = B // NW
C = 32
NCHUNK = BPW // C
NKROW = C * K
NIDX_W = 128
NIDX_R = NKROW // NIDX_W
SCORE_W = 24


def _sc_scores(target, context, noise2d, in_embed, out_embed):
    mesh = plsc.VectorSubcoreMesh(core_axis_name="c", subcore_axis_name="s")

    @functools.partial(
        pl.kernel,
        mesh=mesh,
        compiler_params=pltpu.CompilerParams(
            needs_layout_passes=False, use_tc_tiling_on_sc=False),
        out_type=jax.ShapeDtypeStruct((B * SCORE_W,), jnp.float32),
        scratch_types=[
            pltpu.VMEM((C,), jnp.int32),
            pltpu.VMEM((C,), jnp.int32),
            pltpu.VMEM((NIDX_R, NIDX_W), jnp.int32),
            pltpu.VMEM((C, D), jnp.float32),
            pltpu.VMEM((C, D), jnp.float32),
            pltpu.VMEM((NKROW, D), jnp.float32),
            pltpu.VMEM((C * SCORE_W,), jnp.float32),
            pltpu.SemaphoreType.DMA,
        ],
    )
    def kern(tgt_hbm, ctx_hbm, noi_hbm, inemb_hbm, outemb_hbm, out_hbm,
             tidx, cidx, nidx, trows, crows, nrows, oscr, sem):
        wid = lax.axis_index("s") * 2 + lax.axis_index("c")
        iot = lax.iota(jnp.int32, 16)
        zero16 = jnp.zeros((16,), jnp.int32)

        def chunk_body(ci, carry):
            base = wid * BPW + ci * C
            pltpu.sync_copy(tgt_hbm.at[pl.ds(base, C)], tidx)
            pltpu.sync_copy(ctx_hbm.at[pl.ds(base, C)], cidx)
            for i in range(NIDX_R):
                pltpu.sync_copy(
                    noi_hbm.at[pl.ds(base * K + i * NIDX_W, NIDX_W)],
                    nidx.at[i])
            cps = []
            cps.append(pltpu.async_copy(inemb_hbm.at[tidx], trows, sem))
            cps.append(pltpu.async_copy(outemb_hbm.at[cidx], crows, sem))
            for i in range(NIDX_R):
                cps.append(pltpu.async_copy(
                    outemb_hbm.at[nidx.at[i]],
                    nrows.at[pl.ds(i * NIDX_W, NIDX_W)], sem))
            for cp in cps:
                cp.wait()

            for g in range(C // 16):
                lane_b = g * 16 + iot
                lane_nk = lane_b * K

                def dbody(dd, accs, lane_nk=lane_nk, lane_b=lane_b):
                    dv = jnp.full((16,), 0, jnp.int32) + dd
                    t = plsc.load_gather(trows, [lane_b, dv])
                    c = plsc.load_gather(crows, [lane_b, dv])
                    new = [accs[0] + t * c]
                    for k in range(K):
                        n = plsc.load_gather(nrows, [lane_nk + k, dv])
                        new.append(accs[1 + k] + t * n)
                    return tuple(new)

                accs = lax.fori_loop(
                    0, D, dbody,
                    tuple(jnp.zeros((16,), jnp.float32) for _ in range(K + 1)))

                ob = lane_b * SCORE_W
                plsc.store_scatter(oscr, [ob], -accs[0])
                for k in range(K):
                    plsc.store_scatter(oscr, [ob + (1 + k)], accs[1 + k])
                pad = jnp.full((16,), -1e30, jnp.float32)
                for k in range(K + 1, SCORE_W):
                    plsc.store_scatter(oscr, [ob + k], pad)

            pltpu.sync_copy(oscr, out_hbm.at[pl.ds(base * SCORE_W, C * SCORE_W)])
            return carry

        lax.fori_loop(0, NCHUNK, chunk_body, 0)

    return kern(target, context, noise2d, in_embed, out_embed)


def _tc_loss(scores2d):
    def body(x_ref, o_ref):
        x = x_ref[...]
        sp = jnp.maximum(x, 0.0) + jnp.log(1.0 + jnp.exp(-jnp.abs(x)))
        o_ref[0, 0] = jnp.sum(sp) * (1.0 / B)

    out = pl.pallas_call(
        body,
        out_shape=jax.ShapeDtypeStruct((1, 1), jnp.float32),
        in_specs=[pl.BlockSpec(scores2d.shape, lambda: (0, 0))],
        out_specs=pl.BlockSpec(memory_space=pltpu.SMEM),
    )(scores2d)
    return out[0, 0]


def kernel(target, context, noise, in_embed, out_embed):
    target = target.astype(jnp.int32)
    context = context.astype(jnp.int32)
    noise_flat = noise.astype(jnp.int32).reshape(B * K)
    scores = _sc_scores(target, context, noise_flat, in_embed, out_embed)
    return _tc_loss(scores.reshape(B * SCORE_W // 128, 128))


# speedup vs baseline: 4.0865x; 1.0448x over previous
import functools

import jax
import jax.numpy as jnp
from jax import lax
from jax.experimental import pallas as pl
from jax.experimental.pallas import tpu as pltpu
from jax.experimental.pallas import tpu_sc as plsc

V = 1000000
D = 64
B = 16384
K = 20

NW = 32
BPW = B // NW
C = 32
NCHUNK = BPW // C
NKROW = C * K
NIDX_W = 128
NIDX_R = NKROW // NIDX_W
SCORE_W = 24
DUNROLL = 8


def _sc_scores(target, context, noise2d, in_embed, out_embed):
    mesh = plsc.VectorSubcoreMesh(core_axis_name="c", subcore_axis_name="s")

    @functools.partial(
        pl.kernel,
        mesh=mesh,
        compiler_params=pltpu.CompilerParams(
            needs_layout_passes=False, use_tc_tiling_on_sc=False),
        out_type=jax.ShapeDtypeStruct((B * SCORE_W,), jnp.float32),
        scratch_types=[
            pltpu.VMEM((BPW,), jnp.int32),
            pltpu.VMEM((BPW,), jnp.int32),
            pltpu.VMEM((BPW * K // NIDX_W, NIDX_W), jnp.int32),
            pltpu.VMEM((2, C, D), jnp.float32),
            pltpu.VMEM((2, C, D), jnp.float32),
            pltpu.VMEM((2, NKROW, D), jnp.float32),
            pltpu.VMEM((BPW * SCORE_W,), jnp.float32),
            pltpu.SemaphoreType.DMA,
            pltpu.SemaphoreType.DMA,
        ],
    )
    def kern(tgt_hbm, ctx_hbm, noi_hbm, inemb_hbm, outemb_hbm, out_hbm,
             tidx, cidx, nidx, trows, crows, nrows, oscr, sem0, sem1):
        wid = lax.axis_index("s") * 2 + lax.axis_index("c")
        iot = lax.iota(jnp.int32, 16)
        sems = (sem0, sem1)

        base0 = wid * BPW
        pltpu.sync_copy(tgt_hbm.at[pl.ds(base0, BPW)], tidx)
        pltpu.sync_copy(ctx_hbm.at[pl.ds(base0, BPW)], cidx)
        nr = BPW * K // NIDX_W
        pltpu.sync_copy(noi_hbm.at[pl.ds(wid * nr, nr)], nidx)

        def copies(ci, buf):
            sem = sems[buf]
            cps = [
                pltpu.make_async_copy(
                    inemb_hbm.at[tidx.at[pl.ds(ci * C, C)]], trows.at[buf], sem),
                pltpu.make_async_copy(
                    outemb_hbm.at[cidx.at[pl.ds(ci * C, C)]], crows.at[buf], sem),
            ]
            for i in range(NIDX_R):
                cps.append(pltpu.make_async_copy(
                    outemb_hbm.at[nidx.at[ci * NIDX_R + i]],
                    nrows.at[buf, pl.ds(i * NIDX_W, NIDX_W)], sem))
            return cps

        def fire(ci, buf):
            for cp in copies(ci, buf):
                cp.start()

        def drain(ci, buf):
            for cp in copies(ci, buf):
                cp.wait()

        def compute(ci, buf):
            tr, cr, nr_ = trows.at[buf], crows.at[buf], nrows.at[buf]
            for g in range(C // 16):
                lane_b = g * 16 + iot
                lane_nk = lane_b * K
                ob = (ci * C + lane_b) * SCORE_W

                for kstart, with_pos in ((0, True), (K // 2, False)):
                    nacc = K // 2 + (1 if with_pos else 0)

                    def dbody(d8, accs, lane_nk=lane_nk, lane_b=lane_b,
                              tr=tr, cr=cr, nr_=nr_, kstart=kstart,
                              with_pos=with_pos):
                        accs = list(accs)
                        for du in range(DUNROLL):
                            dv = jnp.full((16,), 0, jnp.int32) + (
                                d8 * DUNROLL + du)
                            t = plsc.load_gather(tr, [lane_b, dv])
                            j = 0
                            if with_pos:
                                c = plsc.load_gather(cr, [lane_b, dv])
                                accs[0] = accs[0] + t * c
                                j = 1
                            for k in range(kstart, kstart + K // 2):
                                n = plsc.load_gather(nr_, [lane_nk + k, dv])
                                accs[j] = accs[j] + t * n
                                j += 1
                        return tuple(accs)

                    accs = lax.fori_loop(
                        0, D // DUNROLL, dbody,
                        tuple(jnp.zeros((16,), jnp.float32)
                              for _ in range(nacc)))

                    j = 0
                    if with_pos:
                        plsc.store_scatter(oscr, [ob], -accs[0])
                        j = 1
                    for k in range(kstart, kstart + K // 2):
                        plsc.store_scatter(oscr, [ob + (1 + k)], accs[j])
                        j += 1

                pad = jnp.full((16,), -1e30, jnp.float32)
                for k in range(K + 1, SCORE_W):
                    plsc.store_scatter(oscr, [ob + k], pad)

        fire(0, 0)

        def jbody(j, carry):
            a = 2 * j
            fire(a + 1, 1)
            drain(a, 0)
            compute(a, 0)
            fire(jnp.minimum(a + 2, NCHUNK - 1), 0)
            drain(a + 1, 1)
            compute(a + 1, 1)
            return carry

        lax.fori_loop(0, NCHUNK // 2, jbody, 0)
        drain(NCHUNK - 1, 0)

        pltpu.sync_copy(oscr, out_hbm.at[pl.ds(base0 * SCORE_W, BPW * SCORE_W)])

    return kern(target, context, noise2d, in_embed, out_embed)


def _tc_loss(scores2d):
    def body(x_ref, o_ref):
        x = x_ref[...]
        sp = jnp.maximum(x, 0.0) + jnp.log(1.0 + jnp.exp(-jnp.abs(x)))
        o_ref[0, 0] = jnp.sum(sp) * (1.0 / B)

    out = pl.pallas_call(
        body,
        out_shape=jax.ShapeDtypeStruct((1, 1), jnp.float32),
        in_specs=[pl.BlockSpec(scores2d.shape, lambda: (0, 0))],
        out_specs=pl.BlockSpec(memory_space=pltpu.SMEM),
    )(scores2d)
    return out[0, 0]


def kernel(target, context, noise, in_embed, out_embed):
    target = target.astype(jnp.int32)
    context = context.astype(jnp.int32)
    noise2d = noise.astype(jnp.int32).reshape(B * K // NIDX_W, NIDX_W)
    scores = _sc_scores(target, context, noise2d, in_embed, out_embed)
    return _tc_loss(scores.reshape(B * SCORE_W // 128, 128))


# speedup vs baseline: 5.2188x; 1.2771x over previous
import functools

import jax
import jax.numpy as jnp
from jax import lax
from jax.experimental import pallas as pl
from jax.experimental.pallas import tpu as pltpu
from jax.experimental.pallas import tpu_sc as plsc

V = 1000000
D = 64
B = 16384
K = 20

NW = 32
BPW = B // NW
C = 32
NCHUNK = BPW // C
NKROW = C * K
NIDX_W = 128
NIDX_R = NKROW // NIDX_W
SCORE_W = 25
DUNROLL = 8


def _sc_scores(target, context, noise2d, in_embed, out_embed):
    mesh = plsc.VectorSubcoreMesh(core_axis_name="c", subcore_axis_name="s")

    @functools.partial(
        pl.kernel,
        mesh=mesh,
        compiler_params=pltpu.CompilerParams(
            needs_layout_passes=False, use_tc_tiling_on_sc=False),
        out_type=jax.ShapeDtypeStruct((B * SCORE_W,), jnp.float32),
        scratch_types=[
            pltpu.VMEM((BPW,), jnp.int32),
            pltpu.VMEM((BPW,), jnp.int32),
            pltpu.VMEM((BPW * K // NIDX_W, NIDX_W), jnp.int32),
            pltpu.VMEM((2, C, D), jnp.float32),
            pltpu.VMEM((2, C, D), jnp.float32),
            pltpu.VMEM((2, NKROW, D), jnp.float32),
            pltpu.VMEM((BPW * SCORE_W,), jnp.float32),
            pltpu.SemaphoreType.DMA,
            pltpu.SemaphoreType.DMA,
        ],
    )
    def kern(tgt_hbm, ctx_hbm, noi_hbm, inemb_hbm, outemb_hbm, out_hbm,
             tidx, cidx, nidx, trows, crows, nrows, oscr, sem0, sem1):
        wid = lax.axis_index("s") * 2 + lax.axis_index("c")
        iot = lax.iota(jnp.int32, 16)
        sems = (sem0, sem1)

        base0 = wid * BPW
        pltpu.sync_copy(tgt_hbm.at[pl.ds(base0, BPW)], tidx)
        pltpu.sync_copy(ctx_hbm.at[pl.ds(base0, BPW)], cidx)
        nr = BPW * K // NIDX_W
        pltpu.sync_copy(noi_hbm.at[pl.ds(wid * nr, nr)], nidx)

        def copies(ci, buf):
            sem = sems[buf]
            cps = [
                pltpu.make_async_copy(
                    inemb_hbm.at[tidx.at[pl.ds(ci * C, C)]], trows.at[buf], sem),
                pltpu.make_async_copy(
                    outemb_hbm.at[cidx.at[pl.ds(ci * C, C)]], crows.at[buf], sem),
            ]
            for i in range(NIDX_R):
                cps.append(pltpu.make_async_copy(
                    outemb_hbm.at[nidx.at[ci * NIDX_R + i]],
                    nrows.at[buf, pl.ds(i * NIDX_W, NIDX_W)], sem))
            return cps

        def fire(ci, buf):
            for cp in copies(ci, buf):
                cp.start()

        def drain(ci, buf):
            for cp in copies(ci, buf):
                cp.wait()

        def compute(ci, buf):
            tr, cr, nr_ = trows.at[buf], crows.at[buf], nrows.at[buf]
            for g in range(C // 16):
                lane_b = g * 16 + iot
                lane_nk = lane_b * K
                ob = (ci * C + lane_b) * SCORE_W

                for kstart, with_pos in ((0, True), (K // 2, False)):
                    nacc = K // 2 + (1 if with_pos else 0)

                    def dbody(d8, accs, lane_nk=lane_nk, lane_b=lane_b,
                              tr=tr, cr=cr, nr_=nr_, kstart=kstart,
                              with_pos=with_pos):
                        accs = list(accs)
                        for du in range(DUNROLL):
                            dv = (iot + (d8 * DUNROLL + du)) & (D - 1)
                            t = plsc.load_gather(tr, [lane_b, dv])
                            j = 0
                            if with_pos:
                                c = plsc.load_gather(cr, [lane_b, dv])
                                accs[0] = accs[0] + t * c
                                j = 1
                            for k in range(kstart, kstart + K // 2):
                                n = plsc.load_gather(nr_, [lane_nk + k, dv])
                                accs[j] = accs[j] + t * n
                                j += 1
                        return tuple(accs)

                    accs = lax.fori_loop(
                        0, D // DUNROLL, dbody,
                        tuple(jnp.zeros((16,), jnp.float32)
                              for _ in range(nacc)))

                    j = 0
                    if with_pos:
                        plsc.store_scatter(oscr, [ob], -accs[0])
                        j = 1
                    for k in range(kstart, kstart + K // 2):
                        plsc.store_scatter(oscr, [ob + (1 + k)], accs[j])
                        j += 1

                pad = jnp.full((16,), -1e30, jnp.float32)
                for k in range(K + 1, SCORE_W):
                    plsc.store_scatter(oscr, [ob + k], pad)

        fire(0, 0)

        def jbody(j, carry):
            a = 2 * j
            fire(a + 1, 1)
            drain(a, 0)
            compute(a, 0)
            fire(jnp.minimum(a + 2, NCHUNK - 1), 0)
            drain(a + 1, 1)
            compute(a + 1, 1)
            return carry

        lax.fori_loop(0, NCHUNK // 2, jbody, 0)
        drain(NCHUNK - 1, 0)

        pltpu.sync_copy(oscr, out_hbm.at[pl.ds(base0 * SCORE_W, BPW * SCORE_W)])

    return kern(target, context, noise2d, in_embed, out_embed)


def _tc_loss(scores2d):
    def body(x_ref, o_ref):
        x = x_ref[...]
        sp = jnp.maximum(x, 0.0) + jnp.log(1.0 + jnp.exp(-jnp.abs(x)))
        o_ref[0, 0] = jnp.sum(sp) * (1.0 / B)

    out = pl.pallas_call(
        body,
        out_shape=jax.ShapeDtypeStruct((1, 1), jnp.float32),
        in_specs=[pl.BlockSpec(scores2d.shape, lambda: (0, 0))],
        out_specs=pl.BlockSpec(memory_space=pltpu.SMEM),
    )(scores2d)
    return out[0, 0]


def kernel(target, context, noise, in_embed, out_embed):
    target = target.astype(jnp.int32)
    context = context.astype(jnp.int32)
    noise2d = noise.astype(jnp.int32).reshape(B * K // NIDX_W, NIDX_W)
    scores = _sc_scores(target, context, noise2d, in_embed, out_embed)
    return _tc_loss(scores.reshape(B * SCORE_W // 128, 128))


# speedup vs baseline: 5.5897x; 1.0711x over previous
import functools

import jax
import jax.numpy as jnp
from jax import lax
from jax.experimental import pallas as pl
from jax.experimental.pallas import tpu as pltpu
from jax.experimental.pallas import tpu_sc as plsc

V = 1000000
D = 64
DP = 128
B = 16384
K = 20

NW = 32
BPW = B // NW
C = 16
NCHUNK = BPW // C
NKROW = C * K
NIDX_W = 64
NIDX_R = NKROW // NIDX_W
SCORE_W = 25
DUNROLL = 8


def _sc_scores(target, context, noise2d, in_embed, out_embed):
    mesh = plsc.VectorSubcoreMesh(core_axis_name="c", subcore_axis_name="s")

    @functools.partial(
        pl.kernel,
        mesh=mesh,
        compiler_params=pltpu.CompilerParams(
            needs_layout_passes=False, use_tc_tiling_on_sc=True),
        out_type=jax.ShapeDtypeStruct((B * SCORE_W,), jnp.float32),
        scratch_types=[
            pltpu.VMEM((BPW,), jnp.int32),
            pltpu.VMEM((BPW,), jnp.int32),
            pltpu.VMEM((BPW * K // NIDX_W, NIDX_W), jnp.int32),
            pltpu.VMEM((2, C, DP), jnp.float32),
            pltpu.VMEM((2, C, DP), jnp.float32),
            pltpu.VMEM((2, NKROW, DP), jnp.float32),
            pltpu.VMEM((BPW * SCORE_W,), jnp.float32),
            pltpu.SemaphoreType.DMA,
            pltpu.SemaphoreType.DMA,
        ],
    )
    def kern(tgt_hbm, ctx_hbm, noi_hbm, inemb_hbm, outemb_hbm, out_hbm,
             tidx, cidx, nidx, trows, crows, nrows, oscr, sem0, sem1):
        wid = lax.axis_index("s") * 2 + lax.axis_index("c")
        iot = lax.iota(jnp.int32, 16)
        sems = (sem0, sem1)

        base0 = wid * BPW
        pltpu.sync_copy(tgt_hbm.at[pl.ds(base0, BPW)], tidx)
        pltpu.sync_copy(ctx_hbm.at[pl.ds(base0, BPW)], cidx)
        nr = BPW * K // NIDX_W
        pltpu.sync_copy(noi_hbm.at[pl.ds(wid * nr, nr)], nidx)

        def copies(ci, buf):
            sem = sems[buf]
            cps = [
                pltpu.make_async_copy(
                    inemb_hbm.at[tidx.at[pl.ds(ci * C, C)]], trows.at[buf], sem),
                pltpu.make_async_copy(
                    outemb_hbm.at[cidx.at[pl.ds(ci * C, C)]], crows.at[buf], sem),
            ]
            for i in range(NIDX_R):
                cps.append(pltpu.make_async_copy(
                    outemb_hbm.at[nidx.at[ci * NIDX_R + i]],
                    nrows.at[buf, pl.ds(i * NIDX_W, NIDX_W)], sem))
            return cps

        def fire(ci, buf):
            for cp in copies(ci, buf):
                cp.start()

        def drain(ci, buf):
            for cp in copies(ci, buf):
                cp.wait()

        def compute(ci, buf):
            tr, cr, nr_ = trows.at[buf], crows.at[buf], nrows.at[buf]
            lane_nk = iot * K
            ob = (ci * C + iot) * SCORE_W

            for kstart, with_pos in ((0, True), (K // 2, False)):
                nacc = K // 2 + (1 if with_pos else 0)

                def dbody(d8, accs, lane_nk=lane_nk, tr=tr, cr=cr, nr_=nr_,
                          kstart=kstart, with_pos=with_pos):
                    accs = list(accs)
                    for du in range(DUNROLL):
                        dv = (iot + (d8 * DUNROLL + du)) & (D - 1)
                        t = plsc.load_gather(tr, [iot, dv])
                        j = 0
                        if with_pos:
                            c = plsc.load_gather(cr, [iot, dv])
                            accs[0] = accs[0] + t * c
                            j = 1
                        for k in range(kstart, kstart + K // 2):
                            n = plsc.load_gather(nr_, [lane_nk + k, dv])
                            accs[j] = accs[j] + t * n
                            j += 1
                    return tuple(accs)

                accs = lax.fori_loop(
                    0, D // DUNROLL, dbody,
                    tuple(jnp.zeros((16,), jnp.float32) for _ in range(nacc)))

                j = 0
                if with_pos:
                    plsc.store_scatter(oscr, [ob], -accs[0])
                    j = 1
                for k in range(kstart, kstart + K // 2):
                    plsc.store_scatter(oscr, [ob + (1 + k)], accs[j])
                    j += 1

            pad = jnp.full((16,), -1e30, jnp.float32)
            for k in range(K + 1, SCORE_W):
                plsc.store_scatter(oscr, [ob + k], pad)

        fire(0, 0)

        def jbody(j, carry):
            a = 2 * j
            fire(a + 1, 1)
            drain(a, 0)
            compute(a, 0)
            fire(jnp.minimum(a + 2, NCHUNK - 1), 0)
            drain(a + 1, 1)
            compute(a + 1, 1)
            return carry

        lax.fori_loop(0, NCHUNK // 2, jbody, 0)
        drain(NCHUNK - 1, 0)

        pltpu.sync_copy(oscr, out_hbm.at[pl.ds(base0 * SCORE_W, BPW * SCORE_W)])

    return kern(target, context, noise2d, in_embed, out_embed)


def _tc_loss(scores2d):
    def body(x_ref, o_ref):
        x = x_ref[...]
        sp = jnp.maximum(x, 0.0) + jnp.log(1.0 + jnp.exp(-jnp.abs(x)))
        o_ref[0, 0] = jnp.sum(sp) * (1.0 / B)

    out = pl.pallas_call(
        body,
        out_shape=jax.ShapeDtypeStruct((1, 1), jnp.float32),
        in_specs=[pl.BlockSpec(scores2d.shape, lambda: (0, 0))],
        out_specs=pl.BlockSpec(memory_space=pltpu.SMEM),
    )(scores2d)
    return out[0, 0]


def kernel(target, context, noise, in_embed, out_embed):
    target = target.astype(jnp.int32)
    context = context.astype(jnp.int32)
    noise2d = noise.astype(jnp.int32).reshape(B * K // NIDX_W, NIDX_W)
    inp = jnp.pad(in_embed, ((0, 0), (0, DP - D)))
    outp = jnp.pad(out_embed, ((0, 0), (0, DP - D)))
    scores = _sc_scores(target, context, noise2d, inp, outp)
    return _tc_loss(scores.reshape(B * SCORE_W // 128, 128))


# speedup vs baseline: 7.2117x; 1.2902x over previous
import functools

import jax
import jax.numpy as jnp
from jax import lax
from jax.experimental import pallas as pl
from jax.experimental.pallas import tpu as pltpu
from jax.experimental.pallas import tpu_sc as plsc

V = 1000000
D = 64
DP = 128
B = 16384
K = 20

NW = 32
BPW = B // NW
C = 16
NCHUNK = BPW // C
NKROW = C * K
NIDX_W = 64
NIDX_R = NKROW // NIDX_W
SCORE_W = 25
DUNROLL = 8


def _sc_scores(emb_target, context, noise2d, out_embed):
    mesh = plsc.VectorSubcoreMesh(core_axis_name="c", subcore_axis_name="s")

    @functools.partial(
        pl.kernel,
        mesh=mesh,
        compiler_params=pltpu.CompilerParams(
            needs_layout_passes=False, use_tc_tiling_on_sc=True),
        out_type=jax.ShapeDtypeStruct((B * SCORE_W,), jnp.float32),
        scratch_types=[
            pltpu.VMEM((BPW,), jnp.int32),
            pltpu.VMEM((BPW * K // NIDX_W, NIDX_W), jnp.int32),
            pltpu.VMEM((2, C, DP), jnp.float32),
            pltpu.VMEM((2, C, DP), jnp.float32),
            pltpu.VMEM((2, NKROW, DP), jnp.float32),
            pltpu.VMEM((BPW * SCORE_W,), jnp.float32),
            pltpu.SemaphoreType.DMA,
            pltpu.SemaphoreType.DMA,
        ],
    )
    def kern(embt_hbm, ctx_hbm, noi_hbm, outemb_hbm, out_hbm,
             cidx, nidx, trows, crows, nrows, oscr, sem0, sem1):
        wid = lax.axis_index("s") * 2 + lax.axis_index("c")
        iot = lax.iota(jnp.int32, 16)
        sems = (sem0, sem1)

        base0 = wid * BPW
        pltpu.sync_copy(ctx_hbm.at[pl.ds(base0, BPW)], cidx)
        nr = BPW * K // NIDX_W
        pltpu.sync_copy(noi_hbm.at[pl.ds(wid * nr, nr)], nidx)

        def copies(ci, buf):
            sem = sems[buf]
            cps = [
                pltpu.make_async_copy(
                    embt_hbm.at[pl.ds(base0 + ci * C, C)], trows.at[buf], sem),
                pltpu.make_async_copy(
                    outemb_hbm.at[cidx.at[pl.ds(ci * C, C)]], crows.at[buf], sem),
            ]
            for i in range(NIDX_R):
                cps.append(pltpu.make_async_copy(
                    outemb_hbm.at[nidx.at[ci * NIDX_R + i]],
                    nrows.at[buf, pl.ds(i * NIDX_W, NIDX_W)], sem))
            return cps

        def fire(ci, buf):
            for cp in copies(ci, buf):
                cp.start()

        def drain(ci, buf):
            for cp in copies(ci, buf):
                cp.wait()

        def compute(ci, buf):
            tr, cr, nr_ = trows.at[buf], crows.at[buf], nrows.at[buf]
            lane_nk = iot * K
            ob = (ci * C + iot) * SCORE_W

            for kstart, with_pos in ((0, True), (K // 2, False)):
                nacc = K // 2 + (1 if with_pos else 0)

                def dbody(d8, accs, lane_nk=lane_nk, tr=tr, cr=cr, nr_=nr_,
                          kstart=kstart, with_pos=with_pos):
                    accs = list(accs)
                    for du in range(DUNROLL):
                        dv = (iot + (d8 * DUNROLL + du)) & (D - 1)
                        t = plsc.load_gather(tr, [iot, dv])
                        j = 0
                        if with_pos:
                            c = plsc.load_gather(cr, [iot, dv])
                            accs[0] = accs[0] + t * c
                            j = 1
                        for k in range(kstart, kstart + K // 2):
                            n = plsc.load_gather(nr_, [lane_nk + k, dv])
                            accs[j] = accs[j] + t * n
                            j += 1
                    return tuple(accs)

                accs = lax.fori_loop(
                    0, D // DUNROLL, dbody,
                    tuple(jnp.zeros((16,), jnp.float32) for _ in range(nacc)))

                j = 0
                if with_pos:
                    plsc.store_scatter(oscr, [ob], -accs[0])
                    j = 1
                for k in range(kstart, kstart + K // 2):
                    plsc.store_scatter(oscr, [ob + (1 + k)], accs[j])
                    j += 1

            pad = jnp.full((16,), -1e30, jnp.float32)
            for k in range(K + 1, SCORE_W):
                plsc.store_scatter(oscr, [ob + k], pad)

        fire(0, 0)

        def jbody(j, carry):
            a = 2 * j
            fire(a + 1, 1)
            drain(a, 0)
            compute(a, 0)
            fire(jnp.minimum(a + 2, NCHUNK - 1), 0)
            drain(a + 1, 1)
            compute(a + 1, 1)
            return carry

        lax.fori_loop(0, NCHUNK // 2, jbody, 0)
        drain(NCHUNK - 1, 0)

        pltpu.sync_copy(oscr, out_hbm.at[pl.ds(base0 * SCORE_W, BPW * SCORE_W)])

    return kern(emb_target, context, noise2d, out_embed)


def _tc_loss(scores2d):
    def body(x_ref, o_ref):
        x = x_ref[...]
        sp = jnp.maximum(x, 0.0) + jnp.log(1.0 + jnp.exp(-jnp.abs(x)))
        o_ref[0, 0] = jnp.sum(sp) * (1.0 / B)

    out = pl.pallas_call(
        body,
        out_shape=jax.ShapeDtypeStruct((1, 1), jnp.float32),
        in_specs=[pl.BlockSpec(scores2d.shape, lambda: (0, 0))],
        out_specs=pl.BlockSpec(memory_space=pltpu.SMEM),
    )(scores2d)
    return out[0, 0]


def kernel(target, context, noise, in_embed, out_embed):
    target = target.astype(jnp.int32)
    context = context.astype(jnp.int32)
    noise2d = noise.astype(jnp.int32).reshape(B * K // NIDX_W, NIDX_W)
    emb_target = jnp.pad(jnp.take(in_embed, target, axis=0),
                         ((0, 0), (0, DP - D)))
    outp = jnp.pad(out_embed, ((0, 0), (0, DP - D)))
    scores = _sc_scores(emb_target, context, noise2d, outp)
    return _tc_loss(scores.reshape(B * SCORE_W // 128, 128))


# speedup vs baseline: 7.2175x; 1.0008x over previous
import functools

import jax
import jax.numpy as jnp
from jax import lax
from jax.experimental import pallas as pl
from jax.experimental.pallas import tpu as pltpu
from jax.experimental.pallas import tpu_sc as plsc

V = 1000000
D = 64
DP = 128
B = 16384
K = 20

NW = 32
BPW = B // NW
C = 16
NCHUNK = BPW // C
NKROW = C * K
NIDX_W = 64
NIDX_R = NKROW // NIDX_W
SCORE_W = 25
DUNROLL = 8


def _sc_scores(emb_target, context, noise2d, out_embed):
    mesh = plsc.VectorSubcoreMesh(core_axis_name="c", subcore_axis_name="s")

    @functools.partial(
        pl.kernel,
        mesh=mesh,
        compiler_params=pltpu.CompilerParams(
            needs_layout_passes=False, use_tc_tiling_on_sc=True),
        out_type=jax.ShapeDtypeStruct((B * SCORE_W,), jnp.float32),
        scratch_types=[
            pltpu.VMEM((BPW,), jnp.int32),
            pltpu.VMEM((BPW * K // NIDX_W, NIDX_W), jnp.int32),
            pltpu.VMEM((2, C, DP), jnp.float32),
            pltpu.VMEM((2, C, DP), jnp.float32),
            pltpu.VMEM((2, NKROW, DP), jnp.float32),
            pltpu.VMEM((BPW * SCORE_W,), jnp.float32),
            pltpu.SemaphoreType.DMA,
            pltpu.SemaphoreType.DMA,
        ],
    )
    def kern(embt_hbm, ctx_hbm, noi_hbm, outemb_hbm, out_hbm,
             cidx, nidx, trows, crows, nrows, oscr, sem0, sem1):
        wid = lax.axis_index("s") * 2 + lax.axis_index("c")
        iot = lax.iota(jnp.int32, 16)
        sems = (sem0, sem1)

        base0 = wid * BPW
        pltpu.sync_copy(ctx_hbm.at[pl.ds(base0, BPW)], cidx)
        nr = BPW * K // NIDX_W
        pltpu.sync_copy(noi_hbm.at[pl.ds(wid * nr, nr)], nidx)

        def copies(ci, buf):
            sem = sems[buf]
            cps = [
                pltpu.make_async_copy(
                    embt_hbm.at[pl.ds(base0 + ci * C, C)], trows.at[buf], sem),
                pltpu.make_async_copy(
                    outemb_hbm.at[cidx.at[pl.ds(ci * C, C)]], crows.at[buf], sem),
            ]
            for i in range(NIDX_R):
                cps.append(pltpu.make_async_copy(
                    outemb_hbm.at[nidx.at[ci * NIDX_R + i]],
                    nrows.at[buf, pl.ds(i * NIDX_W, NIDX_W)], sem))
            return cps

        def fire(ci, buf):
            for cp in copies(ci, buf):
                cp.start()

        def drain(ci, buf):
            for cp in copies(ci, buf):
                cp.wait()

        def compute(ci, buf):
            tr, cr, nr_ = trows.at[buf], crows.at[buf], nrows.at[buf]
            lane_nk = iot * K
            ob = (ci * C + iot) * SCORE_W

            for kstart, with_pos in ((0, True), (K // 2, False)):
                nacc = K // 2 + (1 if with_pos else 0)

                def dbody(d8, accs, lane_nk=lane_nk, tr=tr, cr=cr, nr_=nr_,
                          kstart=kstart, with_pos=with_pos):
                    accs = list(accs)
                    for du in range(DUNROLL):
                        dv = (iot + (d8 * DUNROLL + du)) & (D - 1)
                        t = plsc.load_gather(tr, [iot, dv])
                        j = 0
                        if with_pos:
                            c = plsc.load_gather(cr, [iot, dv])
                            accs[0] = accs[0] + t * c
                            j = 1
                        for k in range(kstart, kstart + K // 2):
                            n = plsc.load_gather(nr_, [lane_nk + k, dv])
                            accs[j] = accs[j] + t * n
                            j += 1
                    return tuple(accs)

                accs = lax.fori_loop(
                    0, D // DUNROLL, dbody,
                    tuple(jnp.zeros((16,), jnp.float32) for _ in range(nacc)))

                j = 0
                if with_pos:
                    plsc.store_scatter(oscr, [ob], -accs[0])
                    j = 1
                for k in range(kstart, kstart + K // 2):
                    plsc.store_scatter(oscr, [ob + (1 + k)], accs[j])
                    j += 1

            pad = jnp.full((16,), -1e30, jnp.float32)
            for k in range(K + 1, SCORE_W):
                plsc.store_scatter(oscr, [ob + k], pad)

        fire(0, 0)

        def jbody(j, carry):
            a = 2 * j
            fire(a + 1, 1)
            drain(a, 0)
            compute(a, 0)
            fire(jnp.minimum(a + 2, NCHUNK - 1), 0)
            drain(a + 1, 1)
            compute(a + 1, 1)
            return carry

        lax.fori_loop(0, NCHUNK // 2, jbody, 0)
        drain(NCHUNK - 1, 0)

        pltpu.sync_copy(oscr, out_hbm.at[pl.ds(base0 * SCORE_W, BPW * SCORE_W)])

    return kern(emb_target, context, noise2d, out_embed)


def _tc_loss(scores2d):
    def body(x_ref, o_ref):
        x = x_ref[...]
        sp = jnp.maximum(x, 0.0) + jnp.log(1.0 + jnp.exp(-jnp.abs(x)))
        o_ref[0, 0] = jnp.sum(sp) * (1.0 / B)

    out = pl.pallas_call(
        body,
        out_shape=jax.ShapeDtypeStruct((1, 1), jnp.float32),
        in_specs=[pl.BlockSpec(scores2d.shape, lambda: (0, 0))],
        out_specs=pl.BlockSpec(memory_space=pltpu.SMEM),
    )(scores2d)
    return out[0, 0]


def kernel(target, context, noise, in_embed, out_embed):
    target = target.astype(jnp.int32)
    context = context.astype(jnp.int32)
    noise2d = noise.astype(jnp.int32).reshape(B * K // NIDX_W, NIDX_W)
    outp = jnp.pad(out_embed, ((0, 0), (0, DP - D)))
    emb_target = jnp.pad(jnp.take(in_embed, target, axis=0),
                         ((0, 0), (0, DP - D)))
    scores = _sc_scores(emb_target, context, noise2d, outp)
    return _tc_loss(scores.reshape(B * SCORE_W // 128, 128))
